# baseline (speedup 1.0000x reference)
import jax
import jax.numpy as jnp
from jax import lax
from jax.experimental import pallas as pl
from jax.experimental.pallas import tpu as pltpu
from jax.experimental.pallas import tpu_sc as plsc

_B = 8
_N = 8400
_MAX_DET = 150
_SCORE_THR = 0.2
_IOU_THR = 0.7
_L = 16
_NPAD = 8448
_SLICE = 1056
_NCHUNK = _SLICE // _L
_OUT_PAD = 768

_F_VAL, _F_IDX, _F_X1, _F_Y1, _F_X2, _F_Y2, _F_AREA = (i * _L for i in range(7))
_PUB = 128


def _nms_body(boxes_hbm, scores_hbm, out_hbm,
              sv, x1v, y1v, x2v, y2v, areav, pubv, rdv, outv, shared):
    c = lax.axis_index("c")
    s_ax = lax.axis_index("s")
    h = s_ax // 8
    q = s_ax % 8
    base = q * _SLICE
    img0 = c * 4 + 2 * h

    for k in (0, 1):
        img = img0 + k
        pltpu.sync_copy(scores_hbm.at[img, pl.ds(base, _SLICE)], sv.at[k])
        pltpu.sync_copy(boxes_hbm.at[4 * img + 0, pl.ds(base, _SLICE)], x1v.at[k])
        pltpu.sync_copy(boxes_hbm.at[4 * img + 1, pl.ds(base, _SLICE)], y1v.at[k])
        pltpu.sync_copy(boxes_hbm.at[4 * img + 2, pl.ds(base, _SLICE)], x2v.at[k])
        pltpu.sync_copy(boxes_hbm.at[4 * img + 3, pl.ds(base, _SLICE)], y2v.at[k])

    lanes = lax.iota(jnp.int32, 16)
    neg_inf = jnp.float32(-jnp.inf)
    ninf_v = jnp.full((16,), neg_inf, jnp.float32)
    zero_i = jnp.zeros((16,), jnp.int32)

    gather_dnums = lax.GatherDimensionNumbers(
        offset_dims=(), collapsed_slice_dims=(0,), start_index_map=(0,)
    )

    def xlane(v, idx):
        return lax.gather(
            v, idx[:, None], gather_dnums, slice_sizes=(1,),
            mode=lax.GatherScatterMode.PROMISE_IN_BOUNDS,
        )

    def make_pass0(k):
        def pass0(i, carry):
            bval, bidx = carry
            sl = pl.ds(i * _L, _L)
            cx = x1v[k, sl]
            cy = y1v[k, sl]
            w = x2v[k, sl]
            h_ = y2v[k, sl]
            x1 = cx - w * 0.5
            y1 = cy - h_ * 0.5
            x2 = cx + w * 0.5
            y2 = cy + h_ * 0.5
            x1v[k, sl] = x1
            y1v[k, sl] = y1
            x2v[k, sl] = x2
            y2v[k, sl] = y2
            areav[k, sl] = jnp.maximum(x2 - x1, 0.0) * jnp.maximum(y2 - y1, 0.0)
            s = sv[k, sl]
            s = jnp.where(s > _SCORE_THR, s, neg_inf)
            sv[k, sl] = s
            idxs = base + i * _L + lanes
            upd = s > bval
            return jnp.where(upd, s, bval), jnp.where(upd, idxs, bidx)
        return pass0

    carry0 = tuple(
        plsc.parallel_loop(0, _NCHUNK, 1, unroll=6,
                           carry=(ninf_v, zero_i))(make_pass0(k))
        for k in (0, 1)
    )

    def local_winner(bval, bidx):
        lmax = bval
        for sh in (8, 4, 2, 1):
            lmax = jnp.maximum(lmax, xlane(lmax, lanes ^ sh))
        cand = jnp.where(bval == lmax, bidx, jnp.int32(2**31 - 1))
        lidx = cand
        for sh in (8, 4, 2, 1):
            lidx = jnp.minimum(lidx, xlane(lidx, lanes ^ sh))
        return lmax, lidx

    def step(t, carry):
        for k in (0, 1):
            lmax, lidx = local_winner(*carry[k])
            loff = jnp.clip(lidx - base, 0, _SLICE - 1)
            o = k * _PUB
            pubv[pl.ds(o + _F_VAL, _L)] = lmax
            pubv[pl.ds(o + _F_IDX, _L)] = plsc.bitcast(lidx, jnp.float32)
            pubv[pl.ds(o + _F_X1, _L)] = plsc.load_gather(x1v.at[k], [loff])
            pubv[pl.ds(o + _F_Y1, _L)] = plsc.load_gather(y1v.at[k], [loff])
            pubv[pl.ds(o + _F_X2, _L)] = plsc.load_gather(x2v.at[k], [loff])
            pubv[pl.ds(o + _F_Y2, _L)] = plsc.load_gather(y2v.at[k], [loff])
            pubv[pl.ds(o + _F_AREA, _L)] = plsc.load_gather(areav.at[k], [loff])

        buf = (t % 2) * 16
        pltpu.sync_copy(pubv, shared.at[buf + s_ax])
        plsc.subcore_barrier()
        pltpu.sync_copy(shared.at[pl.ds(buf + 8 * h, 8)], rdv)

        def merge(a, bb):
            better = (a[0] > bb[0]) | ((a[0] == bb[0]) & (a[1] < bb[1]))
            return tuple(jnp.where(better, fa, fb) for fa, fb in zip(a, bb))

        new_carry = []
        for k in (0, 1):
            o = k * _PUB

            def cand_r(r, o=o):
                return (rdv[r, pl.ds(o + _F_VAL, _L)],
                        plsc.bitcast(rdv[r, pl.ds(o + _F_IDX, _L)], jnp.int32),
                        rdv[r, pl.ds(o + _F_X1, _L)],
                        rdv[r, pl.ds(o + _F_Y1, _L)],
                        rdv[r, pl.ds(o + _F_X2, _L)],
                        rdv[r, pl.ds(o + _F_Y2, _L)],
                        rdv[r, pl.ds(o + _F_AREA, _L)])

            m = [merge(cand_r(2 * r), cand_r(2 * r + 1)) for r in range(4)]
            m = [merge(m[0], m[1]), merge(m[2], m[3])]
            wval, widx, wx1, wy1, wx2, wy2, warea = merge(m[0], m[1])

            @pl.when(q == 0)
            def _():
                kept = wval > ninf_v
                vals = jnp.where(lanes == 0, wx1,
                       jnp.where(lanes == 1, wy1,
                       jnp.where(lanes == 2, wx2,
                       jnp.where(lanes == 3, wy2, wval))))
                vals = jnp.where(kept, vals, jnp.float32(0.0))
                plsc.store_scatter(
                    outv, [k * _OUT_PAD + t * 5 + lanes], vals, mask=lanes < 5)

            wloc = widx - base
            clr_mask = (lanes == 0) & (wloc >= 0) & (wloc < _SLICE)
            wloc_cl = jnp.clip(wloc, 0, _SLICE - 1)
            k_vec = jnp.full((16,), k, jnp.int32)
            plsc.store_scatter(sv, [k_vec, wloc_cl], ninf_v, mask=clr_mask)

            def supp(i, carry2, k=k, wx1=wx1, wy1=wy1, wx2=wx2, wy2=wy2,
                     warea=warea):
                bval2, bidx2 = carry2
                sl = pl.ds(i * _L, _L)
                ix1 = jnp.maximum(wx1, x1v[k, sl])
                iy1 = jnp.maximum(wy1, y1v[k, sl])
                ix2 = jnp.minimum(wx2, x2v[k, sl])
                iy2 = jnp.minimum(wy2, y2v[k, sl])
                inter = jnp.maximum(ix2 - ix1, 0.0) * jnp.maximum(iy2 - iy1, 0.0)
                iou = inter / (warea + areav[k, sl] - inter + 1e-9)
                s = sv[k, sl]
                s = jnp.where(iou > _IOU_THR, neg_inf, s)
                sv[k, sl] = s
                idxs = base + i * _L + lanes
                upd = s > bval2
                return jnp.where(upd, s, bval2), jnp.where(upd, idxs, bidx2)

            new_carry.append(
                plsc.parallel_loop(0, _NCHUNK, 1, unroll=6,
                                   carry=(ninf_v, zero_i))(supp))
        return tuple(new_carry)

    lax.fori_loop(0, _MAX_DET, step, carry0)

    @pl.when(q == 0)
    def _():
        pltpu.sync_copy(outv.at[pl.ds(0, _OUT_PAD)], out_hbm.at[img0])
        pltpu.sync_copy(outv.at[pl.ds(_OUT_PAD, _OUT_PAD)], out_hbm.at[img0 + 1])


@jax.jit
def _nms_sc(boxes_t, scores):
    mesh = plsc.VectorSubcoreMesh(
        core_axis_name="c", subcore_axis_name="s", num_cores=2, num_subcores=16
    )
    f = pl.kernel(
        _nms_body,
        out_type=jax.ShapeDtypeStruct((_B, _OUT_PAD), jnp.float32),
        mesh=mesh,
        compiler_params=pltpu.CompilerParams(
            needs_layout_passes=False, use_tc_tiling_on_sc=False
        ),
        scratch_types=[
            pltpu.VMEM((2, _SLICE), jnp.float32),
            pltpu.VMEM((2, _SLICE), jnp.float32),
            pltpu.VMEM((2, _SLICE), jnp.float32),
            pltpu.VMEM((2, _SLICE), jnp.float32),
            pltpu.VMEM((2, _SLICE), jnp.float32),
            pltpu.VMEM((2, _SLICE), jnp.float32),
            pltpu.VMEM((2 * _PUB,), jnp.float32),
            pltpu.VMEM((8, 2 * _PUB), jnp.float32),
            pltpu.VMEM((2 * _OUT_PAD,), jnp.float32),
            pltpu.VMEM_SHARED((32, 2 * _PUB), jnp.float32),
        ],
    )
    return f(boxes_t, scores)


def kernel(boxes, scores):
    boxes_t = boxes.transpose(0, 2, 1).reshape(_B * 4, _N)
    boxes_t = jnp.pad(boxes_t, ((0, 0), (0, _NPAD - _N)))
    scores_p = jnp.pad(scores, ((0, 0), (0, _NPAD - _N)))
    out = _nms_sc(boxes_t, scores_p)
    return out[:, : _MAX_DET * 5].reshape(_B, _MAX_DET, 5)


# speedup vs baseline: 1.1959x; 1.1959x over previous
import jax
import jax.numpy as jnp
from jax import lax
from jax.experimental import pallas as pl
from jax.experimental.pallas import tpu as pltpu
from jax.experimental.pallas import tpu_sc as plsc

_B = 8
_N = 8400
_MAX_DET = 150
_SCORE_THR = 0.2
_IOU_THR = 0.7
_L = 16
_NPAD = 8448
_SLICE = 2112
_NCHUNK = _SLICE // _L
_OUT_PAD = 768

_F_VAL, _F_IDX, _F_X1, _F_Y1, _F_X2, _F_Y2, _F_AREA = (i * _L for i in range(7))
_PUB = 128


def _nms_body(boxes_hbm, scores_hbm, out_hbm,
              sv, x1v, y1v, x2v, y2v, areav, pubv, rdv, outv, shared):
    c = lax.axis_index("c")
    s_ax = lax.axis_index("s")
    b = c * 4 + s_ax // 4
    slot = s_ax % 4
    base = slot * _SLICE

    pltpu.sync_copy(scores_hbm.at[b, pl.ds(base, _SLICE)], sv)
    pltpu.sync_copy(boxes_hbm.at[4 * b + 0, pl.ds(base, _SLICE)], x1v)
    pltpu.sync_copy(boxes_hbm.at[4 * b + 1, pl.ds(base, _SLICE)], y1v)
    pltpu.sync_copy(boxes_hbm.at[4 * b + 2, pl.ds(base, _SLICE)], x2v)
    pltpu.sync_copy(boxes_hbm.at[4 * b + 3, pl.ds(base, _SLICE)], y2v)

    lanes = lax.iota(jnp.int32, 16)
    neg_inf = jnp.float32(-jnp.inf)
    ninf_v = jnp.full((16,), neg_inf, jnp.float32)
    zero_i = jnp.zeros((16,), jnp.int32)

    gather_dnums = lax.GatherDimensionNumbers(
        offset_dims=(), collapsed_slice_dims=(0,), start_index_map=(0,)
    )

    def xlane(v, idx):
        return lax.gather(
            v, idx[:, None], gather_dnums, slice_sizes=(1,),
            mode=lax.GatherScatterMode.PROMISE_IN_BOUNDS,
        )

    def pass0(i, carry):
        bval, bidx = carry
        sl = pl.ds(i * _L, _L)
        cx = x1v[sl]
        cy = y1v[sl]
        w = x2v[sl]
        h = y2v[sl]
        x1 = cx - w * 0.5
        y1 = cy - h * 0.5
        x2 = cx + w * 0.5
        y2 = cy + h * 0.5
        x1v[sl] = x1
        y1v[sl] = y1
        x2v[sl] = x2
        y2v[sl] = y2
        areav[sl] = jnp.maximum(x2 - x1, 0.0) * jnp.maximum(y2 - y1, 0.0)
        s = sv[sl]
        s = jnp.where(s > _SCORE_THR, s, neg_inf)
        sv[sl] = s
        idxs = base + i * _L + lanes
        upd = s > bval
        return jnp.where(upd, s, bval), jnp.where(upd, idxs, bidx)

    carry0 = plsc.parallel_loop(0, _NCHUNK, 1, unroll=6,
                                carry=(ninf_v, zero_i))(pass0)

    def step(t, carry):
        bval, bidx = carry
        lmax = bval
        for sh in (8, 4, 2, 1):
            lmax = jnp.maximum(lmax, xlane(lmax, lanes ^ sh))
        cand = jnp.where(bval == lmax, bidx, jnp.int32(2**31 - 1))
        lidx = cand
        for sh in (8, 4, 2, 1):
            lidx = jnp.minimum(lidx, xlane(lidx, lanes ^ sh))
        loff = jnp.clip(lidx - base, 0, _SLICE - 1)
        lx1 = plsc.load_gather(x1v, [loff])
        ly1 = plsc.load_gather(y1v, [loff])
        lx2 = plsc.load_gather(x2v, [loff])
        ly2 = plsc.load_gather(y2v, [loff])
        lar = plsc.load_gather(areav, [loff])

        pubv[pl.ds(_F_VAL, _L)] = lmax
        pubv[pl.ds(_F_IDX, _L)] = plsc.bitcast(lidx, jnp.float32)
        pubv[pl.ds(_F_X1, _L)] = lx1
        pubv[pl.ds(_F_Y1, _L)] = ly1
        pubv[pl.ds(_F_X2, _L)] = lx2
        pubv[pl.ds(_F_Y2, _L)] = ly2
        pubv[pl.ds(_F_AREA, _L)] = lar
        buf = (t % 2) * 16
        pltpu.sync_copy(pubv, shared.at[buf + s_ax])
        plsc.subcore_barrier()
        pltpu.sync_copy(shared.at[pl.ds(buf + 4 * (s_ax // 4), 4)], rdv)

        def cand_r(r):
            return (rdv[r, pl.ds(_F_VAL, _L)],
                    plsc.bitcast(rdv[r, pl.ds(_F_IDX, _L)], jnp.int32),
                    rdv[r, pl.ds(_F_X1, _L)],
                    rdv[r, pl.ds(_F_Y1, _L)],
                    rdv[r, pl.ds(_F_X2, _L)],
                    rdv[r, pl.ds(_F_Y2, _L)],
                    rdv[r, pl.ds(_F_AREA, _L)])

        def merge(a, bb):
            better = (a[0] > bb[0]) | ((a[0] == bb[0]) & (a[1] < bb[1]))
            return tuple(jnp.where(better, fa, fb) for fa, fb in zip(a, bb))

        w01 = merge(cand_r(0), cand_r(1))
        w23 = merge(cand_r(2), cand_r(3))
        wval, widx, wx1, wy1, wx2, wy2, warea = merge(w01, w23)

        @pl.when(slot == 0)
        def _():
            kept = wval > ninf_v
            vals = jnp.where(lanes == 0, wx1,
                   jnp.where(lanes == 1, wy1,
                   jnp.where(lanes == 2, wx2,
                   jnp.where(lanes == 3, wy2, wval))))
            vals = jnp.where(kept, vals, jnp.float32(0.0))
            plsc.store_scatter(outv, [t * 5 + lanes], vals, mask=lanes < 5)

        wloc = widx - base
        clr_mask = (lanes == 0) & (wloc >= 0) & (wloc < _SLICE)
        wloc_cl = jnp.clip(wloc, 0, _SLICE - 1)
        plsc.store_scatter(sv, [wloc_cl], ninf_v, mask=clr_mask)

        def supp(i, carry2):
            bval2, bidx2 = carry2
            sl = pl.ds(i * _L, _L)
            ix1 = jnp.maximum(wx1, x1v[sl])
            iy1 = jnp.maximum(wy1, y1v[sl])
            ix2 = jnp.minimum(wx2, x2v[sl])
            iy2 = jnp.minimum(wy2, y2v[sl])
            inter = jnp.maximum(ix2 - ix1, 0.0) * jnp.maximum(iy2 - iy1, 0.0)
            iou = inter / (warea + areav[sl] - inter + 1e-9)
            s = sv[sl]
            s = jnp.where(iou > _IOU_THR, neg_inf, s)
            sv[sl] = s
            idxs = base + i * _L + lanes
            upd = s > bval2
            return jnp.where(upd, s, bval2), jnp.where(upd, idxs, bidx2)

        return plsc.parallel_loop(0, _NCHUNK, 1, unroll=6,
                                  carry=(ninf_v, zero_i))(supp)

    lax.fori_loop(0, _MAX_DET, step, carry0)

    @pl.when(slot == 0)
    def _():
        pltpu.sync_copy(outv, out_hbm.at[b])


@jax.jit
def _nms_sc(boxes_t, scores):
    mesh = plsc.VectorSubcoreMesh(
        core_axis_name="c", subcore_axis_name="s", num_cores=2, num_subcores=16
    )
    f = pl.kernel(
        _nms_body,
        out_type=jax.ShapeDtypeStruct((_B, _OUT_PAD), jnp.float32),
        mesh=mesh,
        compiler_params=pltpu.CompilerParams(
            needs_layout_passes=False, use_tc_tiling_on_sc=False
        ),
        scratch_types=[
            pltpu.VMEM((_SLICE,), jnp.float32),
            pltpu.VMEM((_SLICE,), jnp.float32),
            pltpu.VMEM((_SLICE,), jnp.float32),
            pltpu.VMEM((_SLICE,), jnp.float32),
            pltpu.VMEM((_SLICE,), jnp.float32),
            pltpu.VMEM((_SLICE,), jnp.float32),
            pltpu.VMEM((_PUB,), jnp.float32),
            pltpu.VMEM((4, _PUB), jnp.float32),
            pltpu.VMEM((_OUT_PAD,), jnp.float32),
            pltpu.VMEM_SHARED((32, _PUB), jnp.float32),
        ],
    )
    return f(boxes_t, scores)


def kernel(boxes, scores):
    boxes_t = boxes.transpose(0, 2, 1).reshape(_B * 4, _N)
    boxes_t = jnp.pad(boxes_t, ((0, 0), (0, _NPAD - _N)))
    scores_p = jnp.pad(scores, ((0, 0), (0, _NPAD - _N)))
    out = _nms_sc(boxes_t, scores_p)
    return out[:, : _MAX_DET * 5].reshape(_B, _MAX_DET, 5)
